# baseline (speedup 1.0000x reference)
import functools

import jax
import jax.numpy as jnp
from jax import lax
from jax.experimental import pallas as pl
from jax.experimental.pallas import tpu as pltpu
from jax.experimental.pallas import tpu_sc as plsc

F = 64
N = F + 1
HID = 4 * F
KIN = N * F * 3
OUT_SIZE = F * F

NSC = 2
NSUB = 16
NW = NSC * NSUB
RPW = 16
R_SC = NW * RPW
R_TC = 3 * HID - R_SC

NPASS = 2
RPP = RPW // NPASS
NCH = KIN // 16

NCHUNK = 16
CR = R_TC // NCHUNK


def _gat_body(hF_ref, hT_ref, WfT_ref, WtT_ref, alF_ref, arF_ref, bF_ref,
              alT_ref, arT_ref, bT_ref, outF_ref, outT_ref):
    def one(h, WT, al, ar, b):
        feat = jnp.dot(h, WT, preferred_element_type=jnp.float32)
        el = feat * al
        er0 = feat[0:1, :] * ar
        e = el + er0
        e = jnp.where(e >= 0.0, e, 0.2 * e)
        m = jnp.max(e, axis=0, keepdims=True)
        w = jnp.exp(e - m)
        s = jnp.sum(w, axis=0, keepdims=True)
        att = jnp.sum(w * feat, axis=0, keepdims=True) / s
        return jnp.concatenate([att, feat[1:, :]], axis=0) + b

    outF_ref[...] = one(hF_ref[...], WfT_ref[...], alF_ref[...], arF_ref[...], bF_ref[...])
    outT_ref[...] = one(hT_ref[...], WtT_ref[...], alT_ref[...], arT_ref[...], bT_ref[...])


def _sc_body(W_hbm, x_hbm, out_hbm, xv, wbuf, yv, tred):
    wid = lax.axis_index("s") * NSC + lax.axis_index("c")
    row0 = wid * RPW
    pltpu.sync_copy(x_hbm, xv)

    lane = lax.iota(jnp.int32, 16)
    y = jnp.zeros((16,), jnp.float32)
    for p in range(NPASS):
        pltpu.sync_copy(W_hbm.at[pl.ds(row0 + p * RPP, RPP), :], wbuf)

        def chunk(j, a):
            xj = xv[pl.ds(j * 16, 16)]
            return tuple(a[r] + wbuf[r, pl.ds(j * 16, 16)] * xj
                         for r in range(RPP))

        accs = lax.fori_loop(
            0, NCH, chunk,
            tuple(jnp.zeros((16,), jnp.float32) for _ in range(RPP)))
        for r in range(RPP):
            v = accs[r]
            for s in (8, 4, 2, 1):
                tred[...] = v
                v = v + plsc.load_gather(tred, [(lane + s) & 15])
            y = jnp.where(lane == p * RPP + r, v, y)
    yv[...] = y
    pltpu.sync_copy(yv, out_hbm.at[pl.ds(row0, RPW)])


def _tc_body(x_ref, Whbm_ref, out_ref, wbuf, sems):
    copies = [
        pltpu.make_async_copy(
            Whbm_ref.at[pl.ds(R_SC + c * CR, CR), :],
            wbuf.at[pl.ds(c * CR, CR), :],
            sems.at[c])
        for c in range(NCHUNK)
    ]
    for cp in copies:
        cp.start()

    x = x_ref[...]
    for c in range(NCHUNK):
        copies[c].wait()
        w = wbuf[pl.ds(c * CR, CR), :]
        out_ref[0, c * CR:(c + 1) * CR] = jnp.sum(w * x, axis=1)


def _ep_body(ysc_ref, ytc_ref, Whh_ref, bih_ref, bhh_ref, h0_ref,
             out_ref, h1_ref):
    gx = jnp.concatenate([ysc_ref[0, :], ytc_ref[0, :]], axis=0) + bih_ref[0, :]
    h0 = h0_ref[...]
    W = Whh_ref[...]
    xr, xz, xn = gx[0:HID], gx[HID:2 * HID], gx[2 * HID:]
    hr = jnp.sum(W[0:HID, :] * h0, axis=1) + bhh_ref[0, 0:HID]
    hz = jnp.sum(W[HID:2 * HID, :] * h0, axis=1) + bhh_ref[0, HID:2 * HID]
    hn = jnp.sum(W[2 * HID:, :] * h0, axis=1) + bhh_ref[0, 2 * HID:]
    r = jax.nn.sigmoid(xr + hr)
    z = jax.nn.sigmoid(xz + hz)
    n = jnp.tanh(xn + r * hn)
    h1 = (1.0 - z) * n + z * h0[0]
    out_ref[...] = jnp.concatenate(
        [h1, jnp.zeros((OUT_SIZE - HID,), jnp.float32)], axis=0)
    h1_ref[0, 0, :] = h1


def kernel(data, hidden, W_feat, al_feat, ar_feat, b_feat,
           W_time, al_time, ar_time, b_time, W_ih, W_hh, b_ih, b_hh):
    f32 = jnp.float32
    z1 = jnp.zeros((1, F), f32)
    hF = jnp.concatenate([z1, data], axis=0)
    hT = jnp.concatenate([z1, data.T], axis=0)

    gat = pl.pallas_call(
        _gat_body,
        out_shape=(jax.ShapeDtypeStruct((N, F), f32),
                   jax.ShapeDtypeStruct((N, F), f32)),
    )
    fRF, fRT = gat(hF, hT, W_feat.T, W_time.T,
                   al_feat.reshape(1, F), ar_feat.reshape(1, F), b_feat.reshape(1, F),
                   al_time.reshape(1, F), ar_time.reshape(1, F), b_time.reshape(1, F))

    x = jnp.stack([hF, fRF, fRT], axis=-1).reshape(1, KIN)
    x1d = x.reshape(KIN)

    mesh = plsc.VectorSubcoreMesh(core_axis_name="c", subcore_axis_name="s")
    sc_matvec = functools.partial(
        pl.kernel,
        mesh=mesh,
        compiler_params=pltpu.CompilerParams(needs_layout_passes=False),
        out_type=jax.ShapeDtypeStruct((R_SC,), f32),
        scratch_types=[pltpu.VMEM((KIN,), f32),
                       pltpu.VMEM((RPP, KIN), f32),
                       pltpu.VMEM((16,), f32),
                       pltpu.VMEM((16,), f32)],
    )(_sc_body)
    y_sc = sc_matvec(W_ih, x1d)

    tc_matvec = pl.pallas_call(
        _tc_body,
        in_specs=[
            pl.BlockSpec(memory_space=pltpu.MemorySpace.VMEM),
            pl.BlockSpec(memory_space=pltpu.MemorySpace.HBM),
        ],
        out_specs=pl.BlockSpec(memory_space=pltpu.MemorySpace.VMEM),
        out_shape=jax.ShapeDtypeStruct((1, R_TC), f32),
        scratch_shapes=[pltpu.VMEM((R_TC, KIN), f32),
                        pltpu.SemaphoreType.DMA((NCHUNK,))],
    )
    y_tc = tc_matvec(x, W_ih)

    epilogue = pl.pallas_call(
        _ep_body,
        out_shape=(jax.ShapeDtypeStruct((OUT_SIZE,), f32),
                   jax.ShapeDtypeStruct((1, 1, HID), f32)),
    )
    out, h1 = epilogue(y_sc.reshape(1, R_SC), y_tc, W_hh,
                       b_ih.reshape(1, 3 * HID), b_hh.reshape(1, 3 * HID),
                       hidden.reshape(1, HID))
    return out, h1


# speedup vs baseline: 1.0266x; 1.0266x over previous
import functools

import jax
import jax.numpy as jnp
from jax import lax
from jax.experimental import pallas as pl
from jax.experimental.pallas import tpu as pltpu
from jax.experimental.pallas import tpu_sc as plsc

F = 64
N = F + 1
HID = 4 * F
KIN = N * F * 3
OUT_SIZE = F * F

NSC = 2
NSUB = 16
NW = NSC * NSUB
RPW = 16
R_SC = NW * RPW
R_TC = 3 * HID - R_SC

NG = 4
RPG = RPW // NG
NCH = KIN // 16

NCHUNK = 16
CR = R_TC // NCHUNK


def _gat_body(hF_ref, hT_ref, WfT_ref, WtT_ref, alF_ref, arF_ref, bF_ref,
              alT_ref, arT_ref, bT_ref, outF_ref, outT_ref):
    def one(h, WT, al, ar, b):
        feat = jnp.dot(h, WT, preferred_element_type=jnp.float32)
        el = feat * al
        er0 = feat[0:1, :] * ar
        e = el + er0
        e = jnp.where(e >= 0.0, e, 0.2 * e)
        m = jnp.max(e, axis=0, keepdims=True)
        w = jnp.exp(e - m)
        s = jnp.sum(w, axis=0, keepdims=True)
        att = jnp.sum(w * feat, axis=0, keepdims=True) / s
        return jnp.concatenate([att, feat[1:, :]], axis=0) + b

    outF_ref[...] = one(hF_ref[...], WfT_ref[...], alF_ref[...], arF_ref[...], bF_ref[...])
    outT_ref[...] = one(hT_ref[...], WtT_ref[...], alT_ref[...], arT_ref[...], bT_ref[...])


def _sc_body(W_hbm, x_hbm, out_hbm, xv, wbuf, yv, tred, sems):
    wid = lax.axis_index("s") * NSC + lax.axis_index("c")
    row0 = wid * RPW
    pltpu.sync_copy(x_hbm, xv)

    copies = [
        pltpu.make_async_copy(
            W_hbm.at[pl.ds(row0 + g * RPG, RPG), :],
            wbuf.at[g % 2],
            sems.at[g % 2])
        for g in range(NG)
    ]
    copies[0].start()

    lane = lax.iota(jnp.int32, 16)
    y = jnp.zeros((16,), jnp.float32)
    for g in range(NG):
        if g + 1 < NG:
            copies[g + 1].start()
        copies[g].wait()
        b = g % 2

        def chunk(j, a):
            xj = xv[pl.ds(j * 16, 16)]
            return tuple(a[r] + wbuf[b, r, pl.ds(j * 16, 16)] * xj
                         for r in range(RPG))

        accs = lax.fori_loop(
            0, NCH, chunk,
            tuple(jnp.zeros((16,), jnp.float32) for _ in range(RPG)))
        for r in range(RPG):
            v = accs[r]
            for s in (8, 4, 2, 1):
                tred[...] = v
                v = v + plsc.load_gather(tred, [(lane + s) & 15])
            y = jnp.where(lane == g * RPG + r, v, y)
    yv[...] = y
    pltpu.sync_copy(yv, out_hbm.at[pl.ds(row0, RPW)])


def _tc_body(x_ref, Whbm_ref, out_ref, wbuf, sems):
    copies = [
        pltpu.make_async_copy(
            Whbm_ref.at[pl.ds(R_SC + c * CR, CR), :],
            wbuf.at[pl.ds(c * CR, CR), :],
            sems.at[c])
        for c in range(NCHUNK)
    ]
    for cp in copies:
        cp.start()

    x = x_ref[...]
    for c in range(NCHUNK):
        copies[c].wait()
        w = wbuf[pl.ds(c * CR, CR), :]
        out_ref[0, c * CR:(c + 1) * CR] = jnp.sum(w * x, axis=1)


def _ep_body(ysc_ref, ytc_ref, Whh_ref, bih_ref, bhh_ref, h0_ref,
             out_ref, h1_ref):
    gx = jnp.concatenate([ysc_ref[0, :], ytc_ref[0, :]], axis=0) + bih_ref[0, :]
    h0 = h0_ref[...]
    W = Whh_ref[...]
    xr, xz, xn = gx[0:HID], gx[HID:2 * HID], gx[2 * HID:]
    hr = jnp.sum(W[0:HID, :] * h0, axis=1) + bhh_ref[0, 0:HID]
    hz = jnp.sum(W[HID:2 * HID, :] * h0, axis=1) + bhh_ref[0, HID:2 * HID]
    hn = jnp.sum(W[2 * HID:, :] * h0, axis=1) + bhh_ref[0, 2 * HID:]
    r = jax.nn.sigmoid(xr + hr)
    z = jax.nn.sigmoid(xz + hz)
    n = jnp.tanh(xn + r * hn)
    h1 = (1.0 - z) * n + z * h0[0]
    out_ref[...] = jnp.concatenate(
        [h1, jnp.zeros((OUT_SIZE - HID,), jnp.float32)], axis=0)
    h1_ref[0, 0, :] = h1


def kernel(data, hidden, W_feat, al_feat, ar_feat, b_feat,
           W_time, al_time, ar_time, b_time, W_ih, W_hh, b_ih, b_hh):
    f32 = jnp.float32
    z1 = jnp.zeros((1, F), f32)
    hF = jnp.concatenate([z1, data], axis=0)
    hT = jnp.concatenate([z1, data.T], axis=0)

    gat = pl.pallas_call(
        _gat_body,
        out_shape=(jax.ShapeDtypeStruct((N, F), f32),
                   jax.ShapeDtypeStruct((N, F), f32)),
    )
    fRF, fRT = gat(hF, hT, W_feat.T, W_time.T,
                   al_feat.reshape(1, F), ar_feat.reshape(1, F), b_feat.reshape(1, F),
                   al_time.reshape(1, F), ar_time.reshape(1, F), b_time.reshape(1, F))

    x = jnp.stack([hF, fRF, fRT], axis=-1).reshape(1, KIN)
    x1d = x.reshape(KIN)

    mesh = plsc.VectorSubcoreMesh(core_axis_name="c", subcore_axis_name="s")
    sc_matvec = functools.partial(
        pl.kernel,
        mesh=mesh,
        compiler_params=pltpu.CompilerParams(needs_layout_passes=False),
        out_type=jax.ShapeDtypeStruct((R_SC,), f32),
        scratch_types=[pltpu.VMEM((KIN,), f32),
                       pltpu.VMEM((2, RPG, KIN), f32),
                       pltpu.VMEM((16,), f32),
                       pltpu.VMEM((16,), f32),
                       pltpu.SemaphoreType.DMA((2,))],
    )(_sc_body)
    y_sc = sc_matvec(W_ih, x1d)

    tc_matvec = pl.pallas_call(
        _tc_body,
        in_specs=[
            pl.BlockSpec(memory_space=pltpu.MemorySpace.VMEM),
            pl.BlockSpec(memory_space=pltpu.MemorySpace.HBM),
        ],
        out_specs=pl.BlockSpec(memory_space=pltpu.MemorySpace.VMEM),
        out_shape=jax.ShapeDtypeStruct((1, R_TC), f32),
        scratch_shapes=[pltpu.VMEM((R_TC, KIN), f32),
                        pltpu.SemaphoreType.DMA((NCHUNK,))],
    )
    y_tc = tc_matvec(x, W_ih)

    epilogue = pl.pallas_call(
        _ep_body,
        out_shape=(jax.ShapeDtypeStruct((OUT_SIZE,), f32),
                   jax.ShapeDtypeStruct((1, 1, HID), f32)),
    )
    out, h1 = epilogue(y_sc.reshape(1, R_SC), y_tc, W_hh,
                       b_ih.reshape(1, 3 * HID), b_hh.reshape(1, 3 * HID),
                       hidden.reshape(1, HID))
    return out, h1


# speedup vs baseline: 1.0526x; 1.0254x over previous
import functools

import jax
import jax.numpy as jnp
from jax import lax
from jax.experimental import pallas as pl
from jax.experimental.pallas import tpu as pltpu
from jax.experimental.pallas import tpu_sc as plsc

F = 64
N = F + 1
HID = 4 * F
KIN = N * F * 3
OUT_SIZE = F * F

NSC = 2
NSUB = 16
NW = NSC * NSUB
RPW = 16
R_SC = NW * RPW
R_TC = 3 * HID - R_SC

NG = 4
RPG = RPW // NG
NCH = KIN // 16
UNROLL = 4

NCHUNK = 16
CR = R_TC // NCHUNK


def _gat_body(hF_ref, hT_ref, WfT_ref, WtT_ref, alF_ref, arF_ref, bF_ref,
              alT_ref, arT_ref, bT_ref, outF_ref, outT_ref):
    def one(h, WT, al, ar, b):
        feat = jnp.dot(h, WT, preferred_element_type=jnp.float32)
        el = feat * al
        er0 = feat[0:1, :] * ar
        e = el + er0
        e = jnp.where(e >= 0.0, e, 0.2 * e)
        m = jnp.max(e, axis=0, keepdims=True)
        w = jnp.exp(e - m)
        s = jnp.sum(w, axis=0, keepdims=True)
        att = jnp.sum(w * feat, axis=0, keepdims=True) / s
        return jnp.concatenate([att, feat[1:, :]], axis=0) + b

    outF_ref[...] = one(hF_ref[...], WfT_ref[...], alF_ref[...], arF_ref[...], bF_ref[...])
    outT_ref[...] = one(hT_ref[...], WtT_ref[...], alT_ref[...], arT_ref[...], bT_ref[...])


def _sc_body(W_hbm, x_hbm, out_hbm, xv, wbuf, yv, tred, sems):
    wid = lax.axis_index("s") * NSC + lax.axis_index("c")
    row0 = wid * RPW
    pltpu.sync_copy(x_hbm, xv)

    copies = [
        pltpu.make_async_copy(
            W_hbm.at[pl.ds(row0 + g * RPG, RPG), :],
            wbuf.at[g % 2],
            sems.at[g % 2])
        for g in range(NG)
    ]
    copies[0].start()

    lane = lax.iota(jnp.int32, 16)
    y = jnp.zeros((16,), jnp.float32)
    for g in range(NG):
        if g + 1 < NG:
            copies[g + 1].start()
        copies[g].wait()
        b = g % 2

        def chunk(j, a):
            a = list(a)
            for u in range(UNROLL):
                off = (j * UNROLL + u) * 16
                xj = xv[pl.ds(off, 16)]
                for r in range(RPG):
                    a[r] = a[r] + wbuf[b, r, pl.ds(off, 16)] * xj
            return tuple(a)

        accs = lax.fori_loop(
            0, NCH // UNROLL, chunk,
            tuple(jnp.zeros((16,), jnp.float32) for _ in range(RPG)))
        for r in range(RPG):
            v = accs[r]
            for s in (8, 4, 2, 1):
                tred[...] = v
                v = v + plsc.load_gather(tred, [(lane + s) & 15])
            y = jnp.where(lane == g * RPG + r, v, y)
    yv[...] = y
    pltpu.sync_copy(yv, out_hbm.at[pl.ds(row0, RPW)])


def _tc_body(x_ref, Whbm_ref, out_ref, wbuf, sems):
    copies = [
        pltpu.make_async_copy(
            Whbm_ref.at[pl.ds(R_SC + c * CR, CR), :],
            wbuf.at[pl.ds(c * CR, CR), :],
            sems.at[c])
        for c in range(NCHUNK)
    ]
    for cp in copies:
        cp.start()

    x = x_ref[...]
    for c in range(NCHUNK):
        copies[c].wait()
        w = wbuf[pl.ds(c * CR, CR), :]
        out_ref[0, c * CR:(c + 1) * CR] = jnp.sum(w * x, axis=1)


def _ep_body(ysc_ref, ytc_ref, Whh_ref, bih_ref, bhh_ref, h0_ref,
             out_ref, h1_ref):
    gx = jnp.concatenate([ysc_ref[0, :], ytc_ref[0, :]], axis=0) + bih_ref[0, :]
    h0 = h0_ref[...]
    W = Whh_ref[...]
    xr, xz, xn = gx[0:HID], gx[HID:2 * HID], gx[2 * HID:]
    hr = jnp.sum(W[0:HID, :] * h0, axis=1) + bhh_ref[0, 0:HID]
    hz = jnp.sum(W[HID:2 * HID, :] * h0, axis=1) + bhh_ref[0, HID:2 * HID]
    hn = jnp.sum(W[2 * HID:, :] * h0, axis=1) + bhh_ref[0, 2 * HID:]
    r = jax.nn.sigmoid(xr + hr)
    z = jax.nn.sigmoid(xz + hz)
    n = jnp.tanh(xn + r * hn)
    h1 = (1.0 - z) * n + z * h0[0]
    out_ref[...] = jnp.concatenate(
        [h1, jnp.zeros((OUT_SIZE - HID,), jnp.float32)], axis=0)
    h1_ref[0, 0, :] = h1


def kernel(data, hidden, W_feat, al_feat, ar_feat, b_feat,
           W_time, al_time, ar_time, b_time, W_ih, W_hh, b_ih, b_hh):
    f32 = jnp.float32
    z1 = jnp.zeros((1, F), f32)
    hF = jnp.concatenate([z1, data], axis=0)
    hT = jnp.concatenate([z1, data.T], axis=0)

    gat = pl.pallas_call(
        _gat_body,
        out_shape=(jax.ShapeDtypeStruct((N, F), f32),
                   jax.ShapeDtypeStruct((N, F), f32)),
    )
    fRF, fRT = gat(hF, hT, W_feat.T, W_time.T,
                   al_feat.reshape(1, F), ar_feat.reshape(1, F), b_feat.reshape(1, F),
                   al_time.reshape(1, F), ar_time.reshape(1, F), b_time.reshape(1, F))

    x = jnp.stack([hF, fRF, fRT], axis=-1).reshape(1, KIN)
    x1d = x.reshape(KIN)

    mesh = plsc.VectorSubcoreMesh(core_axis_name="c", subcore_axis_name="s")
    sc_matvec = functools.partial(
        pl.kernel,
        mesh=mesh,
        compiler_params=pltpu.CompilerParams(needs_layout_passes=False),
        out_type=jax.ShapeDtypeStruct((R_SC,), f32),
        scratch_types=[pltpu.VMEM((KIN,), f32),
                       pltpu.VMEM((2, RPG, KIN), f32),
                       pltpu.VMEM((16,), f32),
                       pltpu.VMEM((16,), f32),
                       pltpu.SemaphoreType.DMA((2,))],
    )(_sc_body)
    y_sc = sc_matvec(W_ih, x1d)

    tc_matvec = pl.pallas_call(
        _tc_body,
        in_specs=[
            pl.BlockSpec(memory_space=pltpu.MemorySpace.VMEM),
            pl.BlockSpec(memory_space=pltpu.MemorySpace.HBM),
        ],
        out_specs=pl.BlockSpec(memory_space=pltpu.MemorySpace.VMEM),
        out_shape=jax.ShapeDtypeStruct((1, R_TC), f32),
        scratch_shapes=[pltpu.VMEM((R_TC, KIN), f32),
                        pltpu.SemaphoreType.DMA((NCHUNK,))],
    )
    y_tc = tc_matvec(x, W_ih)

    epilogue = pl.pallas_call(
        _ep_body,
        out_shape=(jax.ShapeDtypeStruct((OUT_SIZE,), f32),
                   jax.ShapeDtypeStruct((1, 1, HID), f32)),
    )
    out, h1 = epilogue(y_sc.reshape(1, R_SC), y_tc, W_hh,
                       b_ih.reshape(1, 3 * HID), b_hh.reshape(1, 3 * HID),
                       hidden.reshape(1, HID))
    return out, h1


# speedup vs baseline: 1.4190x; 1.3481x over previous
import jax
import jax.numpy as jnp
from jax.experimental import pallas as pl
from jax.experimental.pallas import tpu as pltpu

F = 64
N = F + 1
HID = 4 * F
KIN = N * F * 3
OUT_SIZE = F * F
NCHUNK = 16
CR = (3 * HID) // NCHUNK


def _gat_body(hF_ref, hT_ref, WfT_ref, WtT_ref, alF_ref, arF_ref, bF_ref,
              alT_ref, arT_ref, bT_ref, outF_ref, outT_ref):
    def one(h, WT, al, ar, b):
        feat = jnp.dot(h, WT, preferred_element_type=jnp.float32)
        el = feat * al
        er0 = feat[0:1, :] * ar
        e = el + er0
        e = jnp.where(e >= 0.0, e, 0.2 * e)
        m = jnp.max(e, axis=0, keepdims=True)
        w = jnp.exp(e - m)
        s = jnp.sum(w, axis=0, keepdims=True)
        att = jnp.sum(w * feat, axis=0, keepdims=True) / s
        return jnp.concatenate([att, feat[1:, :]], axis=0) + b

    outF_ref[...] = one(hF_ref[...], WfT_ref[...], alF_ref[...], arF_ref[...], bF_ref[...])
    outT_ref[...] = one(hT_ref[...], WtT_ref[...], alT_ref[...], arT_ref[...], bT_ref[...])


def _gru_body(x_ref, Whbm_ref, Whh_ref, bih_ref, bhh_ref, h0_ref,
              out_ref, h1_ref, wbuf, y_scr, sems):
    copies = [
        pltpu.make_async_copy(
            Whbm_ref.at[pl.ds(c * CR, CR), :],
            wbuf.at[pl.ds(c * CR, CR), :],
            sems.at[c])
        for c in range(NCHUNK)
    ]
    for cp in copies:
        cp.start()

    x = x_ref[...]
    for c in range(NCHUNK):
        copies[c].wait()
        w = wbuf[pl.ds(c * CR, CR), :]
        y_scr[0, c * CR:(c + 1) * CR] = jnp.sum(w * x, axis=1)

    h0 = h0_ref[...]
    W = Whh_ref[...]
    gx = y_scr[0, :] + bih_ref[0, :]
    xr, xz, xn = gx[0:HID], gx[HID:2 * HID], gx[2 * HID:]
    hr = jnp.sum(W[0:HID, :] * h0, axis=1) + bhh_ref[0, 0:HID]
    hz = jnp.sum(W[HID:2 * HID, :] * h0, axis=1) + bhh_ref[0, HID:2 * HID]
    hn = jnp.sum(W[2 * HID:, :] * h0, axis=1) + bhh_ref[0, 2 * HID:]
    r = jax.nn.sigmoid(xr + hr)
    z = jax.nn.sigmoid(xz + hz)
    n = jnp.tanh(xn + r * hn)
    h1 = (1.0 - z) * n + z * h0[0]
    out_ref[...] = jnp.concatenate(
        [h1, jnp.zeros((OUT_SIZE - HID,), jnp.float32)], axis=0)
    h1_ref[0, 0, :] = h1


def kernel(data, hidden, W_feat, al_feat, ar_feat, b_feat,
           W_time, al_time, ar_time, b_time, W_ih, W_hh, b_ih, b_hh):
    f32 = jnp.float32
    z1 = jnp.zeros((1, F), f32)
    hF = jnp.concatenate([z1, data], axis=0)
    hT = jnp.concatenate([z1, data.T], axis=0)

    gat = pl.pallas_call(
        _gat_body,
        out_shape=(jax.ShapeDtypeStruct((N, F), f32),
                   jax.ShapeDtypeStruct((N, F), f32)),
    )
    fRF, fRT = gat(hF, hT, W_feat.T, W_time.T,
                   al_feat.reshape(1, F), ar_feat.reshape(1, F), b_feat.reshape(1, F),
                   al_time.reshape(1, F), ar_time.reshape(1, F), b_time.reshape(1, F))

    x = jnp.stack([hF, fRF, fRT], axis=-1).reshape(1, KIN)

    gru = pl.pallas_call(
        _gru_body,
        in_specs=[
            pl.BlockSpec(memory_space=pltpu.MemorySpace.VMEM),
            pl.BlockSpec(memory_space=pltpu.MemorySpace.HBM),
            pl.BlockSpec(memory_space=pltpu.MemorySpace.VMEM),
            pl.BlockSpec(memory_space=pltpu.MemorySpace.VMEM),
            pl.BlockSpec(memory_space=pltpu.MemorySpace.VMEM),
            pl.BlockSpec(memory_space=pltpu.MemorySpace.VMEM),
        ],
        out_specs=(pl.BlockSpec(memory_space=pltpu.MemorySpace.VMEM),
                   pl.BlockSpec(memory_space=pltpu.MemorySpace.VMEM)),
        out_shape=(jax.ShapeDtypeStruct((OUT_SIZE,), f32),
                   jax.ShapeDtypeStruct((1, 1, HID), f32)),
        scratch_shapes=[pltpu.VMEM((3 * HID, KIN), f32),
                        pltpu.VMEM((1, 3 * HID), f32),
                        pltpu.SemaphoreType.DMA((NCHUNK,))],
    )
    out, h1 = gru(x, W_ih, W_hh, b_ih.reshape(1, 3 * HID),
                  b_hh.reshape(1, 3 * HID), hidden.reshape(1, HID))
    return out, h1


# speedup vs baseline: 1.4216x; 1.0018x over previous
import jax
import jax.numpy as jnp
from jax.experimental import pallas as pl
from jax.experimental.pallas import tpu as pltpu

F = 64
N = F + 1
HID = 4 * F
KIN = N * F * 3
OUT_SIZE = F * F
NCHUNK = 16
CR = (3 * HID) // NCHUNK


def _gat_body(hF_ref, hT_ref, WfT_ref, WtT_ref, alF_ref, arF_ref, bF_ref,
              alT_ref, arT_ref, bT_ref, outF_ref, outT_ref):
    def one(h, WT, al, ar, b):
        feat = jnp.dot(h, WT, preferred_element_type=jnp.float32)
        el = feat * al
        er0 = feat[0:1, :] * ar
        e = el + er0
        e = jnp.where(e >= 0.0, e, 0.2 * e)
        m = jnp.max(e, axis=0, keepdims=True)
        w = jnp.exp(e - m)
        s = jnp.sum(w, axis=0, keepdims=True)
        att = jnp.sum(w * feat, axis=0, keepdims=True) / s
        return jnp.concatenate([att, feat[1:, :]], axis=0) + b

    outF_ref[...] = one(hF_ref[...], WfT_ref[...], alF_ref[...], arF_ref[...], bF_ref[...])
    outT_ref[...] = one(hT_ref[...], WtT_ref[...], alT_ref[...], arT_ref[...], bT_ref[...])


def _gru_body(x_ref, Whbm_ref, Whh_ref, bih_ref, bhh_ref, h0_ref,
              out_ref, h1_ref, wbuf, y_scr, whh_buf, sems, hsem):
    copies = [
        pltpu.make_async_copy(
            Whbm_ref.at[pl.ds(c * CR, CR), :],
            wbuf.at[pl.ds(c * CR, CR), :],
            sems.at[c])
        for c in range(NCHUNK)
    ]
    hh_copy = pltpu.make_async_copy(Whh_ref, whh_buf, hsem)
    for cp in copies:
        cp.start()
    hh_copy.start()

    x = x_ref[...]
    for c in range(NCHUNK):
        copies[c].wait()
        w = wbuf[pl.ds(c * CR, CR), :]
        y_scr[0, c * CR:(c + 1) * CR] = jnp.sum(w * x, axis=1)

    hh_copy.wait()
    h0 = h0_ref[...]
    W = whh_buf[...]
    gx = y_scr[0, :] + bih_ref[0, :]
    xr, xz, xn = gx[0:HID], gx[HID:2 * HID], gx[2 * HID:]
    hr = jnp.sum(W[0:HID, :] * h0, axis=1) + bhh_ref[0, 0:HID]
    hz = jnp.sum(W[HID:2 * HID, :] * h0, axis=1) + bhh_ref[0, HID:2 * HID]
    hn = jnp.sum(W[2 * HID:, :] * h0, axis=1) + bhh_ref[0, 2 * HID:]
    r = jax.nn.sigmoid(xr + hr)
    z = jax.nn.sigmoid(xz + hz)
    n = jnp.tanh(xn + r * hn)
    h1 = (1.0 - z) * n + z * h0[0]
    out_ref[...] = jnp.concatenate(
        [h1, jnp.zeros((OUT_SIZE - HID,), jnp.float32)], axis=0)
    h1_ref[0, 0, :] = h1


def kernel(data, hidden, W_feat, al_feat, ar_feat, b_feat,
           W_time, al_time, ar_time, b_time, W_ih, W_hh, b_ih, b_hh):
    f32 = jnp.float32
    z1 = jnp.zeros((1, F), f32)
    hF = jnp.concatenate([z1, data], axis=0)
    hT = jnp.concatenate([z1, data.T], axis=0)

    gat = pl.pallas_call(
        _gat_body,
        out_shape=(jax.ShapeDtypeStruct((N, F), f32),
                   jax.ShapeDtypeStruct((N, F), f32)),
    )
    fRF, fRT = gat(hF, hT, W_feat.T, W_time.T,
                   al_feat.reshape(1, F), ar_feat.reshape(1, F), b_feat.reshape(1, F),
                   al_time.reshape(1, F), ar_time.reshape(1, F), b_time.reshape(1, F))

    x = jnp.stack([hF, fRF, fRT], axis=-1).reshape(1, KIN)

    gru = pl.pallas_call(
        _gru_body,
        in_specs=[
            pl.BlockSpec(memory_space=pltpu.MemorySpace.VMEM),
            pl.BlockSpec(memory_space=pltpu.MemorySpace.HBM),
            pl.BlockSpec(memory_space=pltpu.MemorySpace.HBM),
            pl.BlockSpec(memory_space=pltpu.MemorySpace.VMEM),
            pl.BlockSpec(memory_space=pltpu.MemorySpace.VMEM),
            pl.BlockSpec(memory_space=pltpu.MemorySpace.VMEM),
        ],
        out_specs=(pl.BlockSpec(memory_space=pltpu.MemorySpace.VMEM),
                   pl.BlockSpec(memory_space=pltpu.MemorySpace.VMEM)),
        out_shape=(jax.ShapeDtypeStruct((OUT_SIZE,), f32),
                   jax.ShapeDtypeStruct((1, 1, HID), f32)),
        scratch_shapes=[pltpu.VMEM((3 * HID, KIN), f32),
                        pltpu.VMEM((1, 3 * HID), f32),
                        pltpu.VMEM((3 * HID, HID), f32),
                        pltpu.SemaphoreType.DMA((NCHUNK,)),
                        pltpu.SemaphoreType.DMA],
    )
    out, h1 = gru(x, W_ih, W_hh, b_ih.reshape(1, 3 * HID),
                  b_hh.reshape(1, 3 * HID), hidden.reshape(1, HID))
    return out, h1
